# baseline (speedup 1.0000x reference)
import jax
import jax.numpy as jnp
from jax import lax
from jax.experimental import pallas as pl
from jax.experimental.pallas import tpu as pltpu
from jax.experimental.pallas import tpu_sc as plsc

B = 4096
S = 50
D = 64
L = 16
NC = 2
NS = 16
NW = NC * NS
B_PER_W = B // NW
SP = 64


def kernel(indices, table):
    idx = indices.astype(jnp.int32)
    idx3 = jnp.pad(idx, ((0, 0), (0, SP - S))).reshape(NW, B_PER_W, SP)
    table3 = table.reshape(table.shape[0] // 2, 2, D)
    mesh = plsc.VectorSubcoreMesh(core_axis_name="c", subcore_axis_name="s")

    @pl.kernel(
        out_type=jax.ShapeDtypeStruct((B, D), jnp.float32),
        mesh=mesh,
        scratch_types=[
            pltpu.VMEM((B_PER_W, SP), jnp.int32),
            pltpu.VMEM((B_PER_W, SP), jnp.int32),
            pltpu.VMEM((S, D), jnp.float32),
            pltpu.VMEM((S, D), jnp.float32),
            pltpu.VMEM((B_PER_W, D), jnp.float32),
            pltpu.SemaphoreType.DMA,
            pltpu.SemaphoreType.DMA,
        ],
        compiler_params=pltpu.CompilerParams(use_tc_tiling_on_sc=False),
    )
    def sc_kernel(table_hbm, idx_hbm, out_hbm, hi_v, lo_v, buf_a, buf_b,
                  out_v, sem_a, sem_b):
        wid = lax.axis_index("s") * NC + lax.axis_index("c")
        pltpu.sync_copy(idx_hbm.at[wid], hi_v)
        @pl.loop(0, B_PER_W)
        def _(b):
            for k in range(SP // L):
                sl = pl.ds(k * L, L)
                v = hi_v[b, sl]
                lo_v[b, sl] = v & 1
                hi_v[b, sl] = v >> 1

        def start(b, buf, sem):
            qvecs = [hi_v[b, pl.ds(k * L, L)] for k in range(4)]
            rvecs = [lo_v[b, pl.ds(k * L, L)] for k in range(4)]
            for j in range(S):
                q = qvecs[j // L][j % L]
                rr = rvecs[j // L][j % L]
                pltpu.async_copy(table_hbm.at[q, rr], buf.at[j], sem)

        def wait(buf, sem):
            for j in range(S):
                pltpu.make_async_copy(
                    table_hbm.at[0, 0], buf.at[j], sem).wait()

        def accumulate(buf, b):
            accs = [None] * (D // L)
            for r in range(S):
                for d in range(D // L):
                    sl = pl.ds(d * L, L)
                    v = buf[r, sl]
                    accs[d] = v if accs[d] is None else accs[d] + v
            for d in range(D // L):
                sl = pl.ds(d * L, L)
                out_v[b, sl] = accs[d] * (1.0 / S)

        start(0, buf_a, sem_a)

        @pl.loop(0, B_PER_W // 2)
        def _(g):
            b = 2 * g
            start(b + 1, buf_b, sem_b)
            wait(buf_a, sem_a)
            accumulate(buf_a, b)
            @pl.when(b + 2 < B_PER_W)
            def _():
                start(b + 2, buf_a, sem_a)
            wait(buf_b, sem_b)
            accumulate(buf_b, b + 1)

        pltpu.sync_copy(out_v, out_hbm.at[pl.ds(wid * B_PER_W, B_PER_W)])

    return sc_kernel(table3, idx3)


# speedup vs baseline: 1.0066x; 1.0066x over previous
import jax
import jax.numpy as jnp
from jax import lax
from jax.experimental import pallas as pl
from jax.experimental.pallas import tpu as pltpu
from jax.experimental.pallas import tpu_sc as plsc

B = 4096
S = 50
D = 64
L = 16
NC = 2
NS = 16
NW = NC * NS
B_PER_W = B // NW
CHUNK_B = 2
IDX_PER_CHUNK = CHUNK_B * S
N_CHUNKS = B_PER_W // CHUNK_B


def kernel(indices, table):
    idx = indices.astype(jnp.int32)
    hi3 = (idx >> 1).reshape(NW, N_CHUNKS, IDX_PER_CHUNK)
    lo3 = jnp.pad((idx & 1).reshape(NW, N_CHUNKS, IDX_PER_CHUNK),
                  ((0, 0), (0, 0), (0, 112 - IDX_PER_CHUNK)))
    table3 = table.reshape(table.shape[0] // 2, 2, D)
    mesh = plsc.VectorSubcoreMesh(core_axis_name="c", subcore_axis_name="s")

    @pl.kernel(
        out_type=jax.ShapeDtypeStruct((B, D), jnp.float32),
        mesh=mesh,
        scratch_types=[
            pltpu.VMEM((N_CHUNKS, IDX_PER_CHUNK), jnp.int32),
            pltpu.VMEM((N_CHUNKS, 112), jnp.int32),
            pltpu.VMEM((IDX_PER_CHUNK, 2, D), jnp.float32),
            pltpu.VMEM((IDX_PER_CHUNK, 2, D), jnp.float32),
            pltpu.VMEM((B_PER_W, D), jnp.float32),
            pltpu.SemaphoreType.DMA,
            pltpu.SemaphoreType.DMA,
        ],
        compiler_params=pltpu.CompilerParams(use_tc_tiling_on_sc=False),
    )
    def sc_kernel(table_hbm, hi_hbm, lo_hbm, out_hbm, hi_v, lo_v,
                  buf_a, buf_b, out_v, sem_a, sem_b):
        wid = lax.axis_index("s") * NC + lax.axis_index("c")
        pltpu.sync_copy(hi_hbm.at[wid], hi_v)
        pltpu.sync_copy(lo_hbm.at[wid], lo_v)

        def start(ci, buf, sem):
            src = table_hbm.at[hi_v.at[ci]]
            pltpu.make_async_copy(src, buf, sem).start()

        def wait(ci, buf, sem):
            src = table_hbm.at[hi_v.at[ci]]
            pltpu.make_async_copy(src, buf, sem).wait()

        def accumulate(buf, ci):
            for b in range(CHUNK_B):
                base = b * S
                rvecs = [lo_v[ci, pl.ds(k * L, L)] for k in range(7)]
                accs = [None] * (D // L)
                for r in range(S):
                    c = base + r
                    rr = rvecs[c // L][c % L]
                    for d in range(D // L):
                        sl = pl.ds(d * L, L)
                        v = buf[c, rr, sl]
                        accs[d] = v if accs[d] is None else accs[d] + v
                for d in range(D // L):
                    sl = pl.ds(d * L, L)
                    out_v[ci * CHUNK_B + b, sl] = accs[d] * (1.0 / S)

        start(0, buf_a, sem_a)

        @pl.loop(0, N_CHUNKS // 2)
        def _(g):
            ci = 2 * g
            start(ci + 1, buf_b, sem_b)
            wait(ci, buf_a, sem_a)
            accumulate(buf_a, ci)
            @pl.when(ci + 2 < N_CHUNKS)
            def _():
                start(ci + 2, buf_a, sem_a)
            wait(ci + 1, buf_b, sem_b)
            accumulate(buf_b, ci + 1)

        pltpu.sync_copy(out_v, out_hbm.at[pl.ds(wid * B_PER_W, B_PER_W)])

    return sc_kernel(table3, hi3, lo3)


# speedup vs baseline: 2.3527x; 2.3374x over previous
import jax
import jax.numpy as jnp
from jax import lax
from jax.experimental import pallas as pl
from jax.experimental.pallas import tpu as pltpu
from jax.experimental.pallas import tpu_sc as plsc

B = 4096
S = 50
H = S // 2
D = 64
L = 16
NC = 2
NS = 16
NW = NC * NS
B_PER_W = B // NW
SP = 64


def kernel(indices, table):
    idx = indices.astype(jnp.int32)
    idx3 = jnp.pad(idx, ((0, 0), (0, SP - S))).reshape(NW, B_PER_W, SP)
    table3 = table.reshape(table.shape[0] // 8, 8, D)
    mesh = plsc.VectorSubcoreMesh(core_axis_name="c", subcore_axis_name="s")

    @pl.kernel(
        out_type=jax.ShapeDtypeStruct((B, D), jnp.float32),
        mesh=mesh,
        scratch_types=[
            pltpu.VMEM((B_PER_W, SP), jnp.int32),
            pltpu.VMEM((H, 8, D), jnp.float32),
            pltpu.VMEM((H, 8, D), jnp.float32),
            pltpu.VMEM((B_PER_W, D), jnp.float32),
            pltpu.SemaphoreType.DMA,
            pltpu.SemaphoreType.DMA,
        ],
        compiler_params=pltpu.CompilerParams(use_tc_tiling_on_sc=True),
    )
    def sc_kernel(table_hbm, idx_hbm, out_hbm, idx_v,
                  buf_a, buf_b, out_v, sem_a, sem_b):
        wid = lax.axis_index("s") * NC + lax.axis_index("c")
        pltpu.sync_copy(idx_hbm.at[wid], idx_v)

        def start(b, phase, buf, sem):
            qvecs = [idx_v[b, pl.ds(k * L, L)] >> 3 for k in range(4)]
            for j in range(H):
                t = phase * H + j
                q = qvecs[t // L][t % L]
                pltpu.async_copy(table_hbm.at[q], buf.at[j], sem)

        def wait(buf, sem):
            pltpu.make_async_copy(table_hbm.at[pl.ds(0, H)], buf, sem).wait()

        def accumulate(buf, b, phase):
            rvecs = [idx_v[b, pl.ds(k * L, L)] & 7 for k in range(4)]
            accs = [None] * (D // L)
            for j in range(H):
                t = phase * H + j
                rr = rvecs[t // L][t % L]
                for d in range(D // L):
                    sl = pl.ds(d * L, L)
                    v = buf[j, rr, sl]
                    accs[d] = v if accs[d] is None else accs[d] + v
            for d in range(D // L):
                sl = pl.ds(d * L, L)
                if phase == 0:
                    out_v[b, sl] = accs[d]
                else:
                    out_v[b, sl] = (out_v[b, sl] + accs[d]) * (1.0 / S)

        start(0, 0, buf_a, sem_a)
        start(0, 1, buf_b, sem_b)

        @pl.loop(0, B_PER_W)
        def _(b):
            wait(buf_a, sem_a)
            accumulate(buf_a, b, 0)
            @pl.when(b < B_PER_W - 1)
            def _():
                start(b + 1, 0, buf_a, sem_a)
            wait(buf_b, sem_b)
            accumulate(buf_b, b, 1)
            @pl.when(b < B_PER_W - 1)
            def _():
                start(b + 1, 1, buf_b, sem_b)

        pltpu.sync_copy(out_v, out_hbm.at[pl.ds(wid * B_PER_W, B_PER_W)])

    return sc_kernel(table3, idx3)


# speedup vs baseline: 2.6156x; 1.1117x over previous
import jax
import jax.numpy as jnp
from jax import lax
from jax.experimental import pallas as pl
from jax.experimental.pallas import tpu as pltpu
from jax.experimental.pallas import tpu_sc as plsc

B = 4096
S = 50
H = S // 2
D = 64
L = 16
NC = 2
NS = 16
NW = NC * NS
B_PER_W = B // NW
HB = B_PER_W // 2
SP = 64


def kernel(indices, table):
    idx = indices.astype(jnp.int32)
    idx3 = jnp.pad(idx, ((0, 0), (0, SP - S))).reshape(NW, B_PER_W, SP)
    table3 = table.reshape(table.shape[0] // 8, 8, D)
    mesh = plsc.VectorSubcoreMesh(core_axis_name="c", subcore_axis_name="s")

    @pl.kernel(
        out_type=jax.ShapeDtypeStruct((B, D), jnp.float32),
        mesh=mesh,
        scratch_types=[
            pltpu.VMEM((B_PER_W, SP), jnp.int32),
            pltpu.VMEM((H, 8, D), jnp.float32),
            pltpu.VMEM((H, 8, D), jnp.float32),
            pltpu.VMEM((H, 8, D), jnp.float32),
            pltpu.VMEM((H, 8, D), jnp.float32),
            pltpu.VMEM((HB, D), jnp.float32),
            pltpu.SemaphoreType.DMA,
            pltpu.SemaphoreType.DMA,
            pltpu.SemaphoreType.DMA,
            pltpu.SemaphoreType.DMA,
        ],
        compiler_params=pltpu.CompilerParams(use_tc_tiling_on_sc=True),
    )
    def sc_kernel(table_hbm, idx_hbm, out_hbm, idx_v,
                  buf_a0, buf_a1, buf_b0, buf_b1, out_v,
                  sem_a0, sem_a1, sem_b0, sem_b1):
        wid = lax.axis_index("s") * NC + lax.axis_index("c")
        pltpu.sync_copy(idx_hbm.at[wid], idx_v)

        def start(b, phase, buf, sem):
            qvecs = [idx_v[b, pl.ds(k * L, L)] >> 3 for k in range(4)]
            for j in range(H):
                t = phase * H + j
                q = qvecs[t // L][t % L]
                pltpu.async_copy(table_hbm.at[q], buf.at[j], sem)

        def wait(buf, sem):
            pltpu.make_async_copy(table_hbm.at[pl.ds(0, H)], buf, sem).wait()

        def accumulate(buf, b, phase):
            rvecs = [idx_v[b, pl.ds(k * L, L)] & 7 for k in range(4)]
            accs = [None] * (D // L)
            for j in range(H):
                t = phase * H + j
                rr = rvecs[t // L][t % L]
                for d in range(D // L):
                    sl = pl.ds(d * L, L)
                    v = buf[j, rr, sl]
                    accs[d] = v if accs[d] is None else accs[d] + v
            bb = b & (HB - 1)
            for d in range(D // L):
                sl = pl.ds(d * L, L)
                if phase == 0:
                    out_v[bb, sl] = accs[d]
                else:
                    out_v[bb, sl] = (out_v[bb, sl] + accs[d]) * (1.0 / S)

        start(0, 0, buf_a0, sem_a0)
        start(0, 1, buf_a1, sem_a1)
        start(1, 0, buf_b0, sem_b0)
        start(1, 1, buf_b1, sem_b1)

        @pl.loop(0, B_PER_W // 2)
        def _(g):
            b0 = 2 * g
            b1 = b0 + 1

            def item(b, phase, buf, sem):
                wait(buf, sem)
                accumulate(buf, b, phase)
                @pl.when(b + 2 < B_PER_W)
                def _():
                    start(b + 2, phase, buf, sem)

            item(b0, 0, buf_a0, sem_a0)
            item(b0, 1, buf_a1, sem_a1)
            item(b1, 0, buf_b0, sem_b0)
            item(b1, 1, buf_b1, sem_b1)
            @pl.when(b1 == HB - 1)
            def _():
                pltpu.sync_copy(out_v, out_hbm.at[pl.ds(wid * B_PER_W, HB)])

        pltpu.sync_copy(out_v, out_hbm.at[pl.ds(wid * B_PER_W + HB, HB)])

    return sc_kernel(table3, idx3)
